# Initial kernel scaffold; baseline (speedup 1.0000x reference)
import functools

import jax
import jax.numpy as jnp
from jax import lax
from jax.experimental import pallas as pl
from jax.experimental.pallas import tpu as pltpu
from jax.experimental.pallas import tpu_sc as plsc

_N_USERS = 20000
_N_ITEMS = 30000
_N = _N_USERS + _N_ITEMS
_D = 64
_HALF = 32
_LAYERS = 3
_E = 800000

_NSUB = 16
_CHUNK = 1024
_IDXW = 128
_KSUB = _CHUNK // _IDXW
_EPS = 51200
_EPAD = _EPS * _NSUB
_NCHUNK = _EPS // _CHUNK
_ROWS_PER_SUB = _N // _NSUB
_ZROWS = 625

_mesh = plsc.VectorSubcoreMesh(core_axis_name="c", subcore_axis_name="s")


@jax.jit
def _sc_propagate(ego0, src, dst, w):
    @functools.partial(
        pl.kernel,
        out_type=jax.ShapeDtypeStruct((_LAYERS, 2, _N, _HALF), jnp.float32),
        mesh=_mesh,
        scratch_types=[
            pltpu.VMEM((_KSUB, _IDXW), jnp.int32),
            pltpu.VMEM((_KSUB, _IDXW), jnp.int32),
            pltpu.VMEM((_KSUB, _IDXW), jnp.float32),
            pltpu.VMEM((_KSUB, _IDXW, _HALF), jnp.float32),
            pltpu.VMEM((_ZROWS, _HALF), jnp.float32),
            pltpu.VMEM_SHARED((_N, _HALF), jnp.float32),
            pltpu.SemaphoreType.DMA,
            pltpu.SemaphoreType.DMA,
        ],
    )
    def k(ego0_hbm, src_hbm, dst_hbm, w_hbm, out_hbm,
          srcb, dstb, wb, rows, zeros, acc, gsem, ssem):
        c = lax.axis_index("c")
        s = lax.axis_index("s")

        @pl.loop(0, _ZROWS)
        def _(i):
            zeros[i, pl.ds(0, 16)] = jnp.zeros((16,), jnp.float32)
            zeros[i, pl.ds(16, 16)] = jnp.zeros((16,), jnp.float32)

        @pl.loop(0, _ROWS_PER_SUB // _ZROWS)
        def _(i):
            pltpu.sync_copy(zeros, acc.at[pl.ds(s * _ROWS_PER_SUB + i * _ZROWS, _ZROWS)])

        plsc.subcore_barrier()

        for layer in range(_LAYERS):
            gsrc = ego0_hbm.at[c] if layer == 0 else out_hbm.at[layer - 1, c]

            @pl.loop(0, _NCHUNK)
            def _(t):
                idxrow = s * (_EPS // _IDXW) + t * _KSUB
                pltpu.sync_copy(src_hbm.at[pl.ds(idxrow, _KSUB)], srcb)
                pltpu.sync_copy(dst_hbm.at[pl.ds(idxrow, _KSUB)], dstb)
                pltpu.sync_copy(w_hbm.at[pl.ds(idxrow, _KSUB)], wb)
                gh = [pltpu.async_copy(gsrc.at[srcb.at[j]], rows.at[j], gsem)
                      for j in range(_KSUB)]
                for h in gh:
                    h.wait()
                for j in range(_KSUB):
                    @pl.loop(0, _IDXW)
                    def _(i):
                        wv = wb[j, i]
                        rows[j, i, pl.ds(0, 16)] = rows[j, i, pl.ds(0, 16)] * wv
                        rows[j, i, pl.ds(16, 16)] = rows[j, i, pl.ds(16, 16)] * wv
                sh = [pltpu.async_copy(rows.at[j], acc.at[dstb.at[j]], ssem, add=True)
                      for j in range(_KSUB)]
                for h in sh:
                    h.wait()

            plsc.subcore_barrier()

            @pl.loop(0, _ROWS_PER_SUB // _ZROWS)
            def _(i):
                r0 = s * _ROWS_PER_SUB + i * _ZROWS
                pltpu.sync_copy(acc.at[pl.ds(r0, _ZROWS)],
                                out_hbm.at[layer, c, pl.ds(r0, _ZROWS)])
                if layer < _LAYERS - 1:
                    pltpu.sync_copy(zeros, acc.at[pl.ds(r0, _ZROWS)])

            plsc.subcore_barrier()

    return k(ego0, src, dst, w)


_BN = 2000


def _mean_body(ego0_ref, layers_ref, o_ref):
    s0 = ego0_ref[0] + layers_ref[0, 0] + layers_ref[1, 0] + layers_ref[2, 0]
    s1 = ego0_ref[1] + layers_ref[0, 1] + layers_ref[1, 1] + layers_ref[2, 1]
    o_ref[:, 0:_HALF] = s0 * 0.25
    o_ref[:, _HALF:_D] = s1 * 0.25


@jax.jit
def _mean(ego0, layers):
    return pl.pallas_call(
        _mean_body,
        out_shape=jax.ShapeDtypeStruct((_N, _D), jnp.float32),
        grid=(_N // _BN,),
        in_specs=[
            pl.BlockSpec((2, _BN, _HALF), lambda i: (0, i, 0)),
            pl.BlockSpec((_LAYERS, 2, _BN, _HALF), lambda i: (0, 0, i, 0)),
        ],
        out_specs=pl.BlockSpec((_BN, _D), lambda i: (i, 0)),
    )(ego0, layers)


def kernel(edge_index, edge_weight, user_table, item_table):
    ego0 = jnp.concatenate([user_table, item_table], axis=0)
    ego0_split = ego0.reshape(_N, 2, _HALF).transpose(1, 0, 2)
    pad = _EPAD - _E
    src = jnp.pad(edge_index[0], (0, pad)).reshape(_EPAD // _IDXW, _IDXW)
    dst = jnp.pad(edge_index[1], (0, pad)).reshape(_EPAD // _IDXW, _IDXW)
    w = jnp.pad(edge_weight, (0, pad)).reshape(_EPAD // _IDXW, _IDXW)
    layers = _sc_propagate(ego0_split, src, dst, w)
    final = _mean(ego0_split, layers)
    return final[:_N_USERS], final[_N_USERS:]


# speedup vs baseline: 4.9722x; 4.9722x over previous
import functools

import jax
import jax.numpy as jnp
from jax import lax
from jax.experimental import pallas as pl
from jax.experimental.pallas import tpu as pltpu
from jax.experimental.pallas import tpu_sc as plsc

_N_USERS = 20000
_N_ITEMS = 30000
_N = _N_USERS + _N_ITEMS
_D = 64
_HALF = 32
_LAYERS = 3
_E = 800000

_NSUB = 16
_NPAD = 51200
_CHUNK = 512
_IDXW = 128
_KSUB = _CHUNK // _IDXW
_EPS = 51200
_EPAD = _EPS * _NSUB
_NCHUNK = _EPS // _CHUNK
_ROWS_PER_SUB = _NPAD // _NSUB
_ZCOPIES = _ROWS_PER_SUB // _IDXW

_mesh = plsc.VectorSubcoreMesh(core_axis_name="c", subcore_axis_name="s")


@jax.jit
def _sc_propagate(ego0, src, dst, w):
    @functools.partial(
        pl.kernel,
        out_type=jax.ShapeDtypeStruct((_LAYERS, 2, _NPAD, _HALF), jnp.float32),
        mesh=_mesh,
        scratch_types=[
            pltpu.VMEM((_KSUB, _IDXW), jnp.int32),
            pltpu.VMEM((_KSUB, _IDXW), jnp.int32),
            pltpu.VMEM((_KSUB, _IDXW), jnp.float32),
            pltpu.VMEM((_KSUB, _IDXW, _HALF), jnp.float32),
            pltpu.VMEM_SHARED((_NPAD, _HALF), jnp.float32),
            pltpu.SemaphoreType.DMA,
            pltpu.SemaphoreType.DMA,
        ],
        compiler_params=pltpu.CompilerParams(use_tc_tiling_on_sc=False),
    )
    def k(ego0_hbm, src_hbm, dst_hbm, w_hbm, out_hbm,
          srcb, dstb, wb, rows, acc, gsem, ssem):
        c = lax.axis_index("c")
        s = lax.axis_index("s")

        def zero_acc_range():
            @pl.loop(0, _IDXW)
            def _(i):
                rows[0, i, pl.ds(0, 16)] = jnp.zeros((16,), jnp.float32)
                rows[0, i, pl.ds(16, 16)] = jnp.zeros((16,), jnp.float32)

            @pl.loop(0, _ZCOPIES)
            def _(i):
                pltpu.sync_copy(rows.at[0],
                                acc.at[pl.ds(s * _ROWS_PER_SUB + i * _IDXW, _IDXW)])

        zero_acc_range()
        plsc.subcore_barrier()

        for layer in range(_LAYERS):
            gsrc = ego0_hbm.at[c] if layer == 0 else out_hbm.at[layer - 1, c]

            @pl.loop(0, _NCHUNK)
            def _(t):
                idxrow = s * (_EPS // _IDXW) + t * _KSUB
                pltpu.sync_copy(src_hbm.at[pl.ds(idxrow, _KSUB)], srcb)
                pltpu.sync_copy(dst_hbm.at[pl.ds(idxrow, _KSUB)], dstb)
                pltpu.sync_copy(w_hbm.at[pl.ds(idxrow, _KSUB)], wb)
                gh = [pltpu.async_copy(gsrc.at[srcb.at[j]], rows.at[j], gsem)
                      for j in range(_KSUB)]
                for h in gh:
                    h.wait()
                for j in range(_KSUB):
                    @pl.loop(0, _IDXW, step=16)
                    def _(i0):
                        w16 = wb[j, pl.ds(i0, 16)]
                        for e in range(16):
                            wv = w16[e]
                            rows[j, i0 + e, pl.ds(0, 16)] = rows[j, i0 + e, pl.ds(0, 16)] * wv
                            rows[j, i0 + e, pl.ds(16, 16)] = rows[j, i0 + e, pl.ds(16, 16)] * wv
                sh = [pltpu.async_copy(rows.at[j], acc.at[dstb.at[j]], ssem, add=True)
                      for j in range(_KSUB)]
                for h in sh:
                    h.wait()

            plsc.subcore_barrier()

            @pl.loop(0, _ZCOPIES)
            def _(i):
                r0 = s * _ROWS_PER_SUB + i * _IDXW
                pltpu.sync_copy(acc.at[pl.ds(r0, _IDXW)],
                                out_hbm.at[layer, c, pl.ds(r0, _IDXW)])
            if layer < _LAYERS - 1:
                zero_acc_range()

            plsc.subcore_barrier()

    return k(ego0, src, dst, w)


_BN = 2000


def _mean_body(ego0_ref, layers_ref, o_ref):
    s0 = ego0_ref[0] + layers_ref[0, 0] + layers_ref[1, 0] + layers_ref[2, 0]
    s1 = ego0_ref[1] + layers_ref[0, 1] + layers_ref[1, 1] + layers_ref[2, 1]
    o_ref[:, 0:_HALF] = s0 * 0.25
    o_ref[:, _HALF:_D] = s1 * 0.25


@jax.jit
def _mean(ego0, layers):
    return pl.pallas_call(
        _mean_body,
        out_shape=jax.ShapeDtypeStruct((_N, _D), jnp.float32),
        grid=(_N // _BN,),
        in_specs=[
            pl.BlockSpec((2, _BN, _HALF), lambda i: (0, i, 0)),
            pl.BlockSpec((_LAYERS, 2, _BN, _HALF), lambda i: (0, 0, i, 0)),
        ],
        out_specs=pl.BlockSpec((_BN, _D), lambda i: (i, 0)),
    )(ego0, layers)


def kernel(edge_index, edge_weight, user_table, item_table):
    ego0 = jnp.concatenate([user_table, item_table], axis=0)
    ego0 = jnp.pad(ego0, ((0, _NPAD - _N), (0, 0)))
    ego0_split = ego0.reshape(_NPAD, 2, _HALF).transpose(1, 0, 2)
    pad = _EPAD - _E
    src = jnp.pad(edge_index[0], (0, pad)).reshape(_EPAD // _IDXW, _IDXW)
    dst = jnp.pad(edge_index[1], (0, pad)).reshape(_EPAD // _IDXW, _IDXW)
    w = jnp.pad(edge_weight, (0, pad)).reshape(_EPAD // _IDXW, _IDXW)
    layers = _sc_propagate(ego0_split, src, dst, w)
    final = _mean(ego0_split, layers)
    return final[:_N_USERS], final[_N_USERS:]


# speedup vs baseline: 7.0763x; 1.4232x over previous
import functools

import jax
import jax.numpy as jnp
from jax import lax
from jax.experimental import pallas as pl
from jax.experimental.pallas import tpu as pltpu
from jax.experimental.pallas import tpu_sc as plsc

_N_USERS = 20000
_N_ITEMS = 30000
_N = _N_USERS + _N_ITEMS
_D = 64
_HALF = 32
_LAYERS = 3
_E = 800000

_NSUB = 16
_NPAD = 51200
_IDXW = 128
_GRP = 512
_CPG = _GRP // _IDXW
_EPS = 51200
_EPAD = _EPS * _NSUB
_NGRP = _EPS // _GRP
_NCHUNK = _EPS // _IDXW
_ROWS_PER_SUB = _NPAD // _NSUB
_ZCOPIES = _ROWS_PER_SUB // _IDXW

_mesh = plsc.VectorSubcoreMesh(core_axis_name="c", subcore_axis_name="s")


@jax.jit
def _sc_propagate(ego0, edata):
    @functools.partial(
        pl.kernel,
        out_type=jax.ShapeDtypeStruct((_LAYERS, 2, _NPAD, _HALF), jnp.float32),
        mesh=_mesh,
        scratch_types=[
            pltpu.VMEM((2, 3 * _CPG, _IDXW), jnp.int32),
            pltpu.VMEM((4, _IDXW, _HALF), jnp.float32),
            pltpu.VMEM_SHARED((_NPAD, _HALF), jnp.float32),
            pltpu.SemaphoreType.DMA,
            [pltpu.SemaphoreType.DMA] * 4,
            [pltpu.SemaphoreType.DMA] * 4,
        ],
        compiler_params=pltpu.CompilerParams(use_tc_tiling_on_sc=False,
                                             needs_layout_passes=False),
    )
    def k(ego0_hbm, edata_hbm, out_hbm, ibuf, rows, acc, isem, gsems, ssems):
        c = lax.axis_index("c")
        s = lax.axis_index("s")

        def zero_acc_range():
            @pl.loop(0, _IDXW)
            def _(i):
                rows[0, i, pl.ds(0, 16)] = jnp.zeros((16,), jnp.float32)
                rows[0, i, pl.ds(16, 16)] = jnp.zeros((16,), jnp.float32)

            @pl.loop(0, _ZCOPIES)
            def _(i):
                pltpu.sync_copy(rows.at[0],
                                acc.at[pl.ds(s * _ROWS_PER_SUB + i * _IDXW, _IDXW)])

        def scale_chunk(islot, r):
            @pl.loop(0, _IDXW, step=16)
            def _(i0):
                w16 = plsc.bitcast(ibuf[islot, 2 * _CPG + r, pl.ds(i0, 16)],
                                   jnp.float32)
                for e in range(16):
                    wv = w16[e]
                    rows[r, i0 + e, pl.ds(0, 16)] = rows[r, i0 + e, pl.ds(0, 16)] * wv
                    rows[r, i0 + e, pl.ds(16, 16)] = rows[r, i0 + e, pl.ds(16, 16)] * wv

        zero_acc_range()
        plsc.subcore_barrier()

        for layer in range(_LAYERS):
            gsrc = ego0_hbm.at[c] if layer == 0 else out_hbm.at[layer - 1, c]

            def fire_gather(islot, j, slot):
                return pltpu.async_copy(gsrc.at[ibuf.at[islot, j]],
                                        rows.at[slot], gsems[slot])

            def fire_scatter(islot, r):
                return pltpu.async_copy(rows.at[r],
                                        acc.at[ibuf.at[islot, _CPG + r]],
                                        ssems[r], add=True)

            def drain_scatter(slot):
                pltpu.make_async_copy(rows.at[slot], acc.at[pl.ds(0, _IDXW)],
                                      ssems[slot]).wait()

            def wait_gather(slot):
                pltpu.make_async_copy(gsrc.at[pl.ds(0, _IDXW)], rows.at[slot],
                                      gsems[slot]).wait()

            pltpu.sync_copy(edata_hbm.at[s * _NGRP], ibuf.at[0])
            fire_gather(0, 0, 0)
            fire_gather(0, 1, 1)

            @pl.loop(0, _NGRP // 2)
            def _(t2):
                for half in range(2):
                    g = 2 * t2 + half
                    for r in range(_CPG):
                        t = 4 * g + r
                        wait_gather(r)
                        scale_chunk(half, r)
                        fire_scatter(half, r)

                        @pl.when(t >= 2)
                        def _():
                            drain_scatter((r + 2) % 4)

                        if r == 1:
                            @pl.when(g < _NGRP - 1)
                            def _():
                                pltpu.async_copy(edata_hbm.at[s * _NGRP + g + 1],
                                                 ibuf.at[1 - half], isem)

                        if r == 2:
                            @pl.when(g < _NGRP - 1)
                            def _():
                                pltpu.make_async_copy(
                                    edata_hbm.at[s * _NGRP],
                                    ibuf.at[1 - half], isem).wait()

                        @pl.when(t + 2 < _NCHUNK)
                        def _():
                            if r < 2:
                                fire_gather(half, r + 2, (r + 2) % 4)
                            else:
                                fire_gather(1 - half, r - 2, (r + 2) % 4)

            drain_scatter(2)
            drain_scatter(3)

            plsc.subcore_barrier()

            @pl.loop(0, _ZCOPIES)
            def _(i):
                r0 = s * _ROWS_PER_SUB + i * _IDXW
                pltpu.sync_copy(acc.at[pl.ds(r0, _IDXW)],
                                out_hbm.at[layer, c, pl.ds(r0, _IDXW)])
            if layer < _LAYERS - 1:
                zero_acc_range()

            plsc.subcore_barrier()

    return k(ego0, edata)


_BN = 2000


def _mean_body(ego0_ref, layers_ref, o_ref):
    s0 = ego0_ref[0] + layers_ref[0, 0] + layers_ref[1, 0] + layers_ref[2, 0]
    s1 = ego0_ref[1] + layers_ref[0, 1] + layers_ref[1, 1] + layers_ref[2, 1]
    o_ref[:, 0:_HALF] = s0 * 0.25
    o_ref[:, _HALF:_D] = s1 * 0.25


@jax.jit
def _mean(ego0, layers):
    return pl.pallas_call(
        _mean_body,
        out_shape=jax.ShapeDtypeStruct((_N, _D), jnp.float32),
        grid=(_N // _BN,),
        in_specs=[
            pl.BlockSpec((2, _BN, _HALF), lambda i: (0, i, 0)),
            pl.BlockSpec((_LAYERS, 2, _BN, _HALF), lambda i: (0, 0, i, 0)),
        ],
        out_specs=pl.BlockSpec((_BN, _D), lambda i: (i, 0)),
    )(ego0, layers)


def kernel(edge_index, edge_weight, user_table, item_table):
    ego0 = jnp.concatenate([user_table, item_table], axis=0)
    ego0 = jnp.pad(ego0, ((0, _NPAD - _N), (0, 0)))
    ego0_split = ego0.reshape(_NPAD, 2, _HALF).transpose(1, 0, 2)
    pad = _EPAD - _E
    src = jnp.pad(edge_index[0], (0, pad)).reshape(-1, _CPG, _IDXW)
    dst = jnp.pad(edge_index[1], (0, pad)).reshape(-1, _CPG, _IDXW)
    wbits = lax.bitcast_convert_type(
        jnp.pad(edge_weight, (0, pad)), jnp.int32).reshape(-1, _CPG, _IDXW)
    edata = jnp.concatenate([src, dst, wbits], axis=1)
    layers = _sc_propagate(ego0_split, edata)
    final = _mean(ego0_split, layers)
    return final[:_N_USERS], final[_N_USERS:]
